# Initial kernel scaffold; baseline (speedup 1.0000x reference)
import jax
import jax.numpy as jnp
from jax.experimental import pallas as pl


def kernel(codes, emb_table, W, b):
    raise NotImplementedError("write your pallas kernel here")


# speedup vs baseline: 7.8462x; 7.8462x over previous
import functools

import jax
import jax.numpy as jnp
from jax import lax
from jax.experimental import pallas as pl
from jax.experimental.pallas import tpu as pltpu
from jax.experimental.pallas import tpu_sc as plsc

VOCAB = 8193
RQ = 32
L = 8
D = 64

NC, NS, LANES = 2, 16, 16
NW = NC * NS

TB = 64
IDX_ROWS = TB * L // 128


def _table_body(emb_ref, wr_ref, b_ref, out_ref):
    l = pl.program_id(0)
    t = jnp.dot(emb_ref[...], wr_ref[0], preferred_element_type=jnp.float32)
    out_ref[0] = t + b_ref[...] * (l == 0).astype(jnp.float32)


def _build_table(emb_table, wr, b2d):
    return pl.pallas_call(
        _table_body,
        grid=(L,),
        in_specs=[
            pl.BlockSpec((VOCAB, RQ), lambda l: (0, 0)),
            pl.BlockSpec((1, RQ, D), lambda l: (l, 0, 0)),
            pl.BlockSpec((1, D), lambda l: (0, 0)),
        ],
        out_specs=pl.BlockSpec((1, VOCAB, D), lambda l: (l, 0, 0)),
        out_shape=jax.ShapeDtypeStruct((L, VOCAB, D), jnp.float32),
    )(emb_table, wr, b2d).reshape(L * VOCAB, D)


def _make_sc_lookup(tok):
    tpw = tok // NW
    nb = tpw // TB
    mesh = plsc.VectorSubcoreMesh(core_axis_name="c", subcore_axis_name="s")

    @functools.partial(
        pl.kernel,
        out_type=jax.ShapeDtypeStruct((tok, D), jnp.float32),
        mesh=mesh,
        compiler_params=pltpu.CompilerParams(use_tc_tiling_on_sc=False),
        scratch_types=[
            pltpu.VMEM((TB * L,), jnp.int32),
            pltpu.VMEM((IDX_ROWS, 128), jnp.int32),
            pltpu.VMEM((TB * L, D), jnp.float32),
            pltpu.VMEM((TB, D), jnp.float32),
            pltpu.SemaphoreType.DMA,
        ],
    )
    def sc_lookup(table_hbm, codes_hbm, out_hbm, codes_v, idx_v, rows_v,
                  out_v, sem):
        wid = lax.axis_index("s") * NC + lax.axis_index("c")
        slot_off = lax.rem(lax.iota(jnp.int32, LANES), L) * VOCAB

        def block(bi, carry):
            base = wid * tpw + bi * TB
            pltpu.sync_copy(codes_hbm.at[pl.ds(base * L, TB * L)], codes_v)
            for i in range(TB * L // LANES):
                v = codes_v[pl.ds(i * LANES, LANES)] + slot_off
                idx_v[i // 8, pl.ds((i % 8) * LANES, LANES)] = v
            cps = [
                pltpu.async_copy(table_hbm.at[idx_v.at[j]],
                                 rows_v.at[pl.ds(j * 128, 128)], sem)
                for j in range(IDX_ROWS)
            ]
            for cp in cps:
                cp.wait()

            def tok_body(t, c2):
                rb = t * L
                for c in range(D // LANES):
                    a = rows_v[rb, pl.ds(c * LANES, LANES)]
                    for s in range(1, L):
                        a = a + rows_v[rb + s, pl.ds(c * LANES, LANES)]
                    out_v[t, pl.ds(c * LANES, LANES)] = a
                return c2

            lax.fori_loop(0, TB, tok_body, 0)
            pltpu.sync_copy(out_v, out_hbm.at[pl.ds(base, TB)])
            return carry

        lax.fori_loop(0, nb, block, 0)

    return sc_lookup


def kernel(codes, emb_table, W, b):
    Bsz, Nsz, Lsz = codes.shape
    tok = Bsz * Nsz
    wr = W.reshape(D, L, RQ).transpose(1, 2, 0)
    table = _build_table(emb_table, wr, b.reshape(1, D))
    codes_flat = codes.reshape(-1)
    out = _make_sc_lookup(tok)(table, codes_flat)
    return out.reshape(Bsz, Nsz, D)


# speedup vs baseline: 11.1595x; 1.4223x over previous
import functools

import jax
import jax.numpy as jnp
from jax import lax
from jax.experimental import pallas as pl
from jax.experimental.pallas import tpu as pltpu
from jax.experimental.pallas import tpu_sc as plsc

VOCAB = 8193
RQ = 32
L = 8
D = 64

NC, NS, LANES = 2, 16, 16
NW = NC * NS

TB = 64
IDX_ROWS = TB * L // 128


def _table_body(emb_ref, wr_ref, b_ref, out_ref):
    l = pl.program_id(0)
    t = jnp.dot(emb_ref[...], wr_ref[0], preferred_element_type=jnp.float32)
    out_ref[0] = t + b_ref[...] * (l == 0).astype(jnp.float32)


def _build_table(emb_table, wr, b2d):
    return pl.pallas_call(
        _table_body,
        grid=(L,),
        in_specs=[
            pl.BlockSpec((VOCAB, RQ), lambda l: (0, 0)),
            pl.BlockSpec((1, RQ, D), lambda l: (l, 0, 0)),
            pl.BlockSpec((1, D), lambda l: (0, 0)),
        ],
        out_specs=pl.BlockSpec((1, VOCAB, D), lambda l: (l, 0, 0)),
        out_shape=jax.ShapeDtypeStruct((L, VOCAB, D), jnp.float32),
    )(emb_table, wr, b2d).reshape(L * VOCAB, D)


def _make_sc_lookup(tok):
    tpw = tok // NW
    nb = tpw // TB
    mesh = plsc.VectorSubcoreMesh(core_axis_name="c", subcore_axis_name="s")

    @functools.partial(
        pl.kernel,
        out_type=jax.ShapeDtypeStruct((tok // 2, 2 * D), jnp.float32),
        mesh=mesh,
        compiler_params=pltpu.CompilerParams(use_tc_tiling_on_sc=False),
        scratch_types=[
            pltpu.VMEM((2, TB * L), jnp.int32),
            pltpu.VMEM((2, IDX_ROWS, 128), jnp.int32),
            pltpu.VMEM((2, TB * L, D), jnp.float32),
            pltpu.VMEM((2, TB // 2, 2 * D), jnp.float32),
            pltpu.SemaphoreType.DMA,
            pltpu.SemaphoreType.DMA,
            pltpu.SemaphoreType.DMA,
            pltpu.SemaphoreType.DMA,
        ],
    )
    def sc_lookup(table_hbm, codes_hbm, out_hbm, codes_v, idx_v, rows_v,
                  out_v, semc0, semc1, semg0, semg1):
        wid = lax.axis_index("s") * NC + lax.axis_index("c")
        tok0 = wid * tpw
        slot_off = lax.rem(lax.iota(jnp.int32, LANES), L) * VOCAB
        semc = (semc0, semc1)
        semg = (semg0, semg1)

        def codes_copy(b, buf):
            return pltpu.make_async_copy(
                codes_hbm.at[pl.ds((tok0 + b * TB) * L, TB * L)],
                codes_v.at[buf], semc[buf])

        def gather_copies(buf):
            return [
                pltpu.make_async_copy(
                    table_hbm.at[idx_v.at[buf, j]],
                    rows_v.at[buf, pl.ds(j * 128, 128)],
                    semg[buf])
                for j in range(IDX_ROWS)
            ]

        def compute_idx(buf):
            for i in range(TB * L // LANES):
                v = codes_v[buf, pl.ds(i * LANES, LANES)] + slot_off
                idx_v[buf, i // 8, pl.ds((i % 8) * LANES, LANES)] = v

        def accum_store(b, buf):
            def pair(t2, c2):
                for u in range(2):
                    rb = (t2 * 2 + u) * L
                    for c in range(D // LANES):
                        a = rows_v[buf, rb, pl.ds(c * LANES, LANES)]
                        for s in range(1, L):
                            a = a + rows_v[buf, rb + s, pl.ds(c * LANES, LANES)]
                        out_v[buf, t2, pl.ds(u * D + c * LANES, LANES)] = a
                return c2

            lax.fori_loop(0, TB // 2, pair, 0)
            row0 = tok0 // 2 + b * (TB // 2)
            pltpu.sync_copy(out_v.at[buf], out_hbm.at[pl.ds(row0, TB // 2)])

        def phase(b, bufx, bufy):
            bp1 = jnp.minimum(b + 1, nb - 1)
            bp2 = jnp.minimum(b + 2, nb - 1)
            codes_copy(bp1, bufy).wait()
            compute_idx(bufy)
            for cp in gather_copies(bufy):
                cp.start()
            codes_copy(bp2, bufx).start()
            for cp in gather_copies(bufx):
                cp.wait()
            accum_store(b, bufx)

        codes_copy(0, 0).start()
        codes_copy(0, 0).wait()
        compute_idx(0)
        for cp in gather_copies(0):
            cp.start()
        codes_copy(1, 1).start()

        def pair_body(p, carry):
            phase(2 * p, 0, 1)
            phase(2 * p + 1, 1, 0)
            return carry

        lax.fori_loop(0, nb // 2, pair_body, 0)

        for cp in gather_copies(0):
            cp.wait()
        codes_copy(nb - 1, 1).wait()

    return sc_lookup


def kernel(codes, emb_table, W, b):
    Bsz, Nsz, Lsz = codes.shape
    tok = Bsz * Nsz
    wr = W.reshape(D, L, RQ).transpose(1, 2, 0)
    table = _build_table(emb_table, wr, b.reshape(1, D))
    codes_flat = codes.reshape(-1)
    out = _make_sc_lookup(tok)(table, codes_flat)
    return out.reshape(Bsz, Nsz, D)


# speedup vs baseline: 12.3900x; 1.1103x over previous
import functools

import jax
import jax.numpy as jnp
from jax import lax
from jax.experimental import pallas as pl
from jax.experimental.pallas import tpu as pltpu
from jax.experimental.pallas import tpu_sc as plsc

VOCAB = 8193
VPAD = 8208
RQ = 32
L = 8
D = 64

NC, NS, LANES = 2, 16, 16
NW = NC * NS

TB = 128
IDX_ROWS = TB * L // 128


def _table_body(emb_ref, wr_ref, b_ref, out_ref):
    l = pl.program_id(0)
    t = jnp.dot(emb_ref[...], wr_ref[0], preferred_element_type=jnp.float32)
    t = t + b_ref[...] * (l == 0).astype(jnp.float32)
    out_ref[...] = t.astype(jnp.bfloat16)


def _build_table(emb_pad, wr, b2d):
    return pl.pallas_call(
        _table_body,
        grid=(L,),
        in_specs=[
            pl.BlockSpec((VPAD, RQ), lambda l: (0, 0)),
            pl.BlockSpec((1, RQ, D), lambda l: (l, 0, 0)),
            pl.BlockSpec((1, D), lambda l: (0, 0)),
        ],
        out_specs=pl.BlockSpec((VPAD, D), lambda l: (l, 0)),
        out_shape=jax.ShapeDtypeStruct((L * VPAD, D), jnp.bfloat16),
    )(emb_pad, wr, b2d)


def _make_sc_lookup(tok):
    tpw = tok // NW
    nb = tpw // TB
    mesh = plsc.VectorSubcoreMesh(core_axis_name="c", subcore_axis_name="s")

    @functools.partial(
        pl.kernel,
        out_type=jax.ShapeDtypeStruct((tok, D), jnp.bfloat16),
        mesh=mesh,
        compiler_params=pltpu.CompilerParams(use_tc_tiling_on_sc=False),
        scratch_types=[
            pltpu.VMEM((2, TB * L), jnp.int32),
            pltpu.VMEM((2, IDX_ROWS, 128), jnp.int32),
            pltpu.VMEM((2, TB * L, D), jnp.bfloat16),
            pltpu.VMEM((2, TB, D), jnp.bfloat16),
            pltpu.SemaphoreType.DMA,
            pltpu.SemaphoreType.DMA,
            pltpu.SemaphoreType.DMA,
            pltpu.SemaphoreType.DMA,
        ],
    )
    def sc_lookup(table_hbm, codes_hbm, out_hbm, codes_v, idx_v, rows_v,
                  out_v, semc0, semc1, semg0, semg1):
        wid = lax.axis_index("s") * NC + lax.axis_index("c")
        tok0 = wid * tpw
        slot_off = lax.rem(lax.iota(jnp.int32, LANES), L) * VPAD
        semc = (semc0, semc1)
        semg = (semg0, semg1)

        def codes_copy(b, buf):
            return pltpu.make_async_copy(
                codes_hbm.at[pl.ds((tok0 + b * TB) * L, TB * L)],
                codes_v.at[buf], semc[buf])

        def gather_copies(buf):
            return [
                pltpu.make_async_copy(
                    table_hbm.at[idx_v.at[buf, j]],
                    rows_v.at[buf, pl.ds(j * 128, 128)],
                    semg[buf])
                for j in range(IDX_ROWS)
            ]

        def compute_idx(buf):
            for i in range(TB * L // LANES):
                v = codes_v[buf, pl.ds(i * LANES, LANES)] + slot_off
                idx_v[buf, i // 8, pl.ds((i % 8) * LANES, LANES)] = v

        def accum_store(b, buf):
            def tok_body(t, c2):
                rb = t * L
                for h in range(D // 32):
                    a = rows_v[buf, rb, pl.ds(h * 32, 32)]
                    for s in range(1, L):
                        a = a + rows_v[buf, rb + s, pl.ds(h * 32, 32)]
                    out_v[buf, t, pl.ds(h * 32, 32)] = a
                return c2

            lax.fori_loop(0, TB, tok_body, 0)
            pltpu.sync_copy(out_v.at[buf],
                            out_hbm.at[pl.ds(tok0 + b * TB, TB)])

        def phase(b, bufx, bufy):
            bp1 = jnp.minimum(b + 1, nb - 1)
            bp2 = jnp.minimum(b + 2, nb - 1)
            codes_copy(bp1, bufy).wait()
            compute_idx(bufy)
            for cp in gather_copies(bufy):
                cp.start()
            codes_copy(bp2, bufx).start()
            for cp in gather_copies(bufx):
                cp.wait()
            accum_store(b, bufx)

        codes_copy(0, 0).start()
        codes_copy(0, 0).wait()
        compute_idx(0)
        for cp in gather_copies(0):
            cp.start()
        codes_copy(1, 1).start()

        def pair_body(p, carry):
            phase(2 * p, 0, 1)
            phase(2 * p + 1, 1, 0)
            return carry

        lax.fori_loop(0, nb // 2, pair_body, 0)

        for cp in gather_copies(0):
            cp.wait()
        codes_copy(nb - 1, 1).wait()

    return sc_lookup


def kernel(codes, emb_table, W, b):
    Bsz, Nsz, Lsz = codes.shape
    tok = Bsz * Nsz
    wr = W.reshape(D, L, RQ).transpose(1, 2, 0)
    emb_pad = jnp.pad(emb_table, ((0, VPAD - VOCAB), (0, 0)))
    table = _build_table(emb_pad, wr, b.reshape(1, D))
    codes_flat = codes.reshape(-1)
    out = _make_sc_lookup(tok)(table, codes_flat)
    return out.astype(jnp.float32).reshape(Bsz, Nsz, D)


# speedup vs baseline: 15.8779x; 1.2815x over previous
import functools

import jax
import jax.numpy as jnp
from jax import lax
from jax.experimental import pallas as pl
from jax.experimental.pallas import tpu as pltpu
from jax.experimental.pallas import tpu_sc as plsc

VOCAB = 8193
VPAD = 8208
RQ = 32
L = 8
D = 64

NC, NS, LANES = 2, 16, 16
NW = NC * NS

TB = 128
IDX_ROWS = TB * L // 128


def _table_body(emb_ref, wr_ref, b_ref, out_ref):
    l = pl.program_id(0)
    t = jnp.dot(emb_ref[...], wr_ref[0], preferred_element_type=jnp.float32)
    t = t + b_ref[...] * (l == 0).astype(jnp.float32)
    out_ref[...] = t.astype(jnp.bfloat16)


def _build_table(emb_pad, wr, b2d):
    return pl.pallas_call(
        _table_body,
        grid=(L,),
        in_specs=[
            pl.BlockSpec((VPAD, RQ), lambda l: (0, 0)),
            pl.BlockSpec((1, RQ, D), lambda l: (l, 0, 0)),
            pl.BlockSpec((1, D), lambda l: (0, 0)),
        ],
        out_specs=pl.BlockSpec((VPAD, D), lambda l: (l, 0)),
        out_shape=jax.ShapeDtypeStruct((L * VPAD, D), jnp.bfloat16),
    )(emb_pad, wr, b2d)


def _make_sc_lookup(bsz, nsz):
    bt_n = bsz // TB
    nblk = nsz * bt_n
    nb = nblk // NW
    out_len = nsz * L * bt_n * (D // L) * TB * L
    mesh = plsc.VectorSubcoreMesh(core_axis_name="c", subcore_axis_name="s")

    @functools.partial(
        pl.kernel,
        out_type=jax.ShapeDtypeStruct((nsz * D * bsz,), jnp.float32),
        mesh=mesh,
        compiler_params=pltpu.CompilerParams(use_tc_tiling_on_sc=False,
                                             needs_layout_passes=False),
        scratch_types=[
            pltpu.VMEM((2, TB * L), jnp.int32),
            pltpu.VMEM((2, IDX_ROWS, 128), jnp.int32),
            pltpu.VMEM((2, TB * L, D), jnp.bfloat16),
            pltpu.VMEM((2, D * TB), jnp.float32),
            pltpu.SemaphoreType.DMA,
            pltpu.SemaphoreType.DMA,
            pltpu.SemaphoreType.DMA,
            pltpu.SemaphoreType.DMA,
            pltpu.SemaphoreType.DMA,
            pltpu.SemaphoreType.DMA,
        ],
    )
    def sc_lookup(table_hbm, codes_hbm, out_hbm, codes_v, idx_v, rows_v,
                  stage_v, semc0, semc1, semg0, semg1, semo0, semo1):
        wid = lax.axis_index("s") * NC + lax.axis_index("c")
        blk0 = wid * nb
        semc = (semc0, semc1)
        semg = (semg0, semg1)
        semo = (semo0, semo1)
        base = lax.iota(jnp.int32, LANES) * (2 * TB)
        bases = [base + (h * 32 * TB + p * TB)
                 for h in range(D // 32) for p in range(2)]

        def codes_copy(b, buf):
            fb = blk0 + b
            return pltpu.make_async_copy(
                codes_hbm.at[pl.ds(fb * (TB * L), TB * L)],
                codes_v.at[buf], semc[buf])

        def gather_copies(buf):
            return [
                pltpu.make_async_copy(
                    table_hbm.at[idx_v.at[buf, j]],
                    rows_v.at[buf, pl.ds(j * 128, 128)],
                    semg[buf])
                for j in range(IDX_ROWS)
            ]

        def out_copies(b, buf):
            fb = blk0 + b
            n = fb // bt_n
            bt = fb % bt_n
            return [
                pltpu.make_async_copy(
                    stage_v.at[buf, pl.ds(dt * (8 * TB), 8 * TB)],
                    out_hbm.at[pl.ds(((n * L + dt) * bt_n + bt) * (8 * TB),
                                     8 * TB)],
                    semo[buf])
                for dt in range(D // 8)
            ]

        def compute_idx(buf):
            for i in range(TB * L // LANES):
                v = codes_v[buf, pl.ds(i * LANES, LANES)] + (i // 8) * VPAD
                idx_v[buf, i // 8, pl.ds((i % 8) * LANES, LANES)] = v

        def accum_store(b, buf):
            for cp in out_copies(b, buf):
                cp.wait()

            def tok_body(t, c2):
                for h in range(D // 32):
                    a = rows_v[buf, t, pl.ds(h * 32, 32)]
                    for s in range(1, L):
                        a = a + rows_v[buf, s * 128 + t, pl.ds(h * 32, 32)]
                    even, odd = plsc.unpack(a, format=plsc.PackFormat.INTERLEAVED)
                    plsc.store_scatter(stage_v.at[buf], [bases[2 * h] + t], even)
                    plsc.store_scatter(stage_v.at[buf], [bases[2 * h + 1] + t], odd)
                return c2

            lax.fori_loop(0, TB, tok_body, 0)
            for cp in out_copies(b, buf):
                cp.start()

        def phase(b, bufx, bufy):
            bp1 = jnp.minimum(b + 1, nb - 1)
            bp2 = jnp.minimum(b + 2, nb - 1)
            codes_copy(bp1, bufy).wait()
            compute_idx(bufy)
            for cp in gather_copies(bufy):
                cp.start()
            codes_copy(bp2, bufx).start()
            for cp in gather_copies(bufx):
                cp.wait()
            accum_store(b, bufx)

        codes_copy(0, 0).start()
        for buf in (0, 1):
            for cp in out_copies(buf, buf):
                cp.start()
        codes_copy(0, 0).wait()
        compute_idx(0)
        for cp in gather_copies(0):
            cp.start()
        codes_copy(1, 1).start()

        def pair_body(p, carry):
            phase(2 * p, 0, 1)
            phase(2 * p + 1, 1, 0)
            return carry

        lax.fori_loop(0, nb // 2, pair_body, 0)

        for cp in gather_copies(0):
            cp.wait()
        codes_copy(nb - 1, 1).wait()
        for buf in (0, 1):
            for cp in out_copies(nb - 2 + buf, buf):
                cp.wait()

    return sc_lookup


def kernel(codes, emb_table, W, b):
    bsz, nsz, lsz = codes.shape
    wr = W.reshape(D, L, RQ).transpose(1, 2, 0)
    emb_pad = jnp.pad(emb_table, ((0, VPAD - VOCAB), (0, 0)))
    table = _build_table(emb_pad, wr, b.reshape(1, D))
    codes_t = (codes.transpose(1, 2, 0)
               .reshape(nsz, lsz, bsz // TB, TB)
               .transpose(0, 2, 1, 3)
               .reshape(-1))
    out = _make_sc_lookup(bsz, nsz)(table, codes_t)
    out = (out.reshape(nsz, D // 8, bsz // TB, 8, TB)
           .transpose(2, 4, 0, 1, 3)
           .reshape(bsz, nsz, D))
    return out


# speedup vs baseline: 23.5031x; 1.4802x over previous
import functools

import jax
import jax.numpy as jnp
from jax import lax
from jax.experimental import pallas as pl
from jax.experimental.pallas import tpu as pltpu
from jax.experimental.pallas import tpu_sc as plsc

VOCAB = 8193
VPAD = 8208
RQ = 32
L = 8
D = 64

NC, NS, LANES = 2, 16, 16
NW = NC * NS

TB = 128
IDX_ROWS = TB * L // 128


def _table_body(emb_ref, wr_ref, b_ref, out_ref):
    l = pl.program_id(0)
    t = jnp.dot(emb_ref[...], wr_ref[0], preferred_element_type=jnp.float32)
    t = t + b_ref[...] * (l == 0).astype(jnp.float32)
    out_ref[...] = t.astype(jnp.bfloat16)


def _build_table(emb_pad, wr, b2d):
    return pl.pallas_call(
        _table_body,
        grid=(L,),
        in_specs=[
            pl.BlockSpec((VPAD, RQ), lambda l: (0, 0)),
            pl.BlockSpec((1, RQ, D), lambda l: (l, 0, 0)),
            pl.BlockSpec((1, D), lambda l: (0, 0)),
        ],
        out_specs=pl.BlockSpec((VPAD, D), lambda l: (l, 0)),
        out_shape=jax.ShapeDtypeStruct((L * VPAD, D), jnp.bfloat16),
    )(emb_pad, wr, b2d)


def _make_sc_lookup(bsz, nsz):
    bt_n = bsz // TB
    nblk = nsz * bt_n
    nb = nblk // NW
    out_len = nsz * L * bt_n * (D // L) * TB * L
    mesh = plsc.VectorSubcoreMesh(core_axis_name="c", subcore_axis_name="s")

    @functools.partial(
        pl.kernel,
        out_type=jax.ShapeDtypeStruct((nsz * D * bsz,), jnp.float32),
        mesh=mesh,
        compiler_params=pltpu.CompilerParams(use_tc_tiling_on_sc=False,
                                             needs_layout_passes=False),
        scratch_types=[
            pltpu.VMEM((2, TB * L), jnp.int32),
            pltpu.VMEM((2, IDX_ROWS, 128), jnp.int32),
            pltpu.VMEM((2, TB * L, D), jnp.bfloat16),
            pltpu.VMEM((2, D * TB), jnp.float32),
            pltpu.SemaphoreType.DMA,
            pltpu.SemaphoreType.DMA,
            pltpu.SemaphoreType.DMA,
            pltpu.SemaphoreType.DMA,
            pltpu.SemaphoreType.DMA,
            pltpu.SemaphoreType.DMA,
        ],
    )
    def sc_lookup(table_hbm, codes_hbm, out_hbm, codes_v, idx_v, rows_v,
                  stage_v, semc0, semc1, semg0, semg1, semo0, semo1):
        wid = lax.axis_index("s") * NC + lax.axis_index("c")
        blk0 = wid * nb
        semc = (semc0, semc1)
        semg = (semg0, semg1)
        semo = (semo0, semo1)
        base = lax.iota(jnp.int32, LANES) * (2 * TB)
        bases = [base + (h * 32 * TB + p * TB)
                 for h in range(D // 32) for p in range(2)]

        def codes_copy(b, buf):
            fb = blk0 + b
            return pltpu.make_async_copy(
                codes_hbm.at[pl.ds(fb * (TB * L), TB * L)],
                codes_v.at[buf], semc[buf])

        def gather_copies(buf):
            return [
                pltpu.make_async_copy(
                    table_hbm.at[idx_v.at[buf, j]],
                    rows_v.at[buf, pl.ds(j * 128, 128)],
                    semg[buf])
                for j in range(IDX_ROWS)
            ]

        def out_copies(b, buf):
            fb = blk0 + b
            n = fb // bt_n
            bt = fb % bt_n
            return [
                pltpu.make_async_copy(
                    stage_v.at[buf, pl.ds(dt * (8 * TB), 8 * TB)],
                    out_hbm.at[pl.ds(((n * L + dt) * bt_n + bt) * (8 * TB),
                                     8 * TB)],
                    semo[buf])
                for dt in range(D // 8)
            ]

        def compute_idx(buf):
            for i in range(TB * L // LANES):
                v = codes_v[buf, pl.ds(i * LANES, LANES)] + (i // 8) * VPAD
                idx_v[buf, i // 8, pl.ds((i % 8) * LANES, LANES)] = v

        def accum_store(b, buf):
            for cp in out_copies(b, buf):
                cp.wait()

            @plsc.parallel_loop(0, TB, unroll=8)
            def tok_body(t):
                for h in range(D // 32):
                    a = rows_v[buf, t, pl.ds(h * 32, 32)]
                    for s in range(1, L):
                        a = a + rows_v[buf, s * 128 + t, pl.ds(h * 32, 32)]
                    even, odd = plsc.unpack(a, format=plsc.PackFormat.INTERLEAVED)
                    plsc.store_scatter(stage_v.at[buf], [bases[2 * h] + t], even)
                    plsc.store_scatter(stage_v.at[buf], [bases[2 * h + 1] + t], odd)
            for cp in out_copies(b, buf):
                cp.start()

        def phase(b, bufx, bufy):
            bp1 = jnp.minimum(b + 1, nb - 1)
            bp2 = jnp.minimum(b + 2, nb - 1)
            codes_copy(bp1, bufy).wait()
            compute_idx(bufy)
            for cp in gather_copies(bufy):
                cp.start()
            codes_copy(bp2, bufx).start()
            for cp in gather_copies(bufx):
                cp.wait()
            accum_store(b, bufx)

        codes_copy(0, 0).start()
        for buf in (0, 1):
            for cp in out_copies(buf, buf):
                cp.start()
        codes_copy(0, 0).wait()
        compute_idx(0)
        for cp in gather_copies(0):
            cp.start()
        codes_copy(1, 1).start()

        def pair_body(p, carry):
            phase(2 * p, 0, 1)
            phase(2 * p + 1, 1, 0)
            return carry

        lax.fori_loop(0, nb // 2, pair_body, 0)

        for cp in gather_copies(0):
            cp.wait()
        codes_copy(nb - 1, 1).wait()
        for buf in (0, 1):
            for cp in out_copies(nb - 2 + buf, buf):
                cp.wait()

    return sc_lookup


def kernel(codes, emb_table, W, b):
    bsz, nsz, lsz = codes.shape
    wr = W.reshape(D, L, RQ).transpose(1, 2, 0)
    emb_pad = jnp.pad(emb_table, ((0, VPAD - VOCAB), (0, 0)))
    table = _build_table(emb_pad, wr, b.reshape(1, D))
    codes_t = (codes.transpose(1, 2, 0)
               .reshape(nsz, lsz, bsz // TB, TB)
               .transpose(0, 2, 1, 3)
               .reshape(-1))
    out = _make_sc_lookup(bsz, nsz)(table, codes_t)
    out = (out.reshape(nsz, D // 8, bsz // TB, 8, TB)
           .transpose(2, 4, 0, 1, 3)
           .reshape(bsz, nsz, D))
    return out


# speedup vs baseline: 23.6286x; 1.0053x over previous
import functools

import jax
import jax.numpy as jnp
from jax import lax
from jax.experimental import pallas as pl
from jax.experimental.pallas import tpu as pltpu
from jax.experimental.pallas import tpu_sc as plsc

VOCAB = 8193
VPAD = 8208
RQ = 32
L = 8
D = 64

NC, NS, LANES = 2, 16, 16
NW = NC * NS

TB = 128
IDX_ROWS = TB * L // 128


def _table_body(emb_ref, wr_ref, b_ref, out_ref):
    l = pl.program_id(0)
    t = jnp.dot(emb_ref[...], wr_ref[0], preferred_element_type=jnp.float32)
    t = t + b_ref[...] * (l == 0).astype(jnp.float32)
    out_ref[...] = t.astype(jnp.bfloat16)


def _build_table(emb_pad, wr, b2d):
    return pl.pallas_call(
        _table_body,
        grid=(L,),
        in_specs=[
            pl.BlockSpec((VPAD, RQ), lambda l: (0, 0)),
            pl.BlockSpec((1, RQ, D), lambda l: (l, 0, 0)),
            pl.BlockSpec((1, D), lambda l: (0, 0)),
        ],
        out_specs=pl.BlockSpec((VPAD, D), lambda l: (l, 0)),
        out_shape=jax.ShapeDtypeStruct((L * VPAD, D), jnp.bfloat16),
    )(emb_pad, wr, b2d)


def _make_sc_lookup(bsz, nsz):
    bt_n = bsz // TB
    nblk = nsz * bt_n
    nb = nblk // NW
    out_len = nsz * L * bt_n * (D // L) * TB * L
    mesh = plsc.VectorSubcoreMesh(core_axis_name="c", subcore_axis_name="s")

    @functools.partial(
        pl.kernel,
        out_type=jax.ShapeDtypeStruct((nsz * D * bsz,), jnp.float32),
        mesh=mesh,
        compiler_params=pltpu.CompilerParams(use_tc_tiling_on_sc=False,
                                             needs_layout_passes=False),
        scratch_types=[
            pltpu.VMEM((2, TB * L), jnp.int32),
            pltpu.VMEM((2, TB * L), jnp.int32),
            pltpu.VMEM((2, TB * L, D), jnp.bfloat16),
            pltpu.VMEM((2, D * TB), jnp.float32),
            pltpu.SemaphoreType.DMA,
            pltpu.SemaphoreType.DMA,
            pltpu.SemaphoreType.DMA,
            pltpu.SemaphoreType.DMA,
            pltpu.SemaphoreType.DMA,
            pltpu.SemaphoreType.DMA,
        ],
    )
    def sc_lookup(table_hbm, codes_hbm, out_hbm, codes_v, idx_v, rows_v,
                  stage_v, semc0, semc1, semg0, semg1, semo0, semo1):
        wid = lax.axis_index("s") * NC + lax.axis_index("c")
        blk0 = wid * nb
        semc = (semc0, semc1)
        semg = (semg0, semg1)
        semo = (semo0, semo1)
        base = lax.iota(jnp.int32, LANES) * (2 * TB)
        bases = [base + (h * 32 * TB + p * TB)
                 for h in range(D // 32) for p in range(2)]

        def codes_copy(b, buf):
            fb = blk0 + b
            return pltpu.make_async_copy(
                codes_hbm.at[pl.ds(fb * (TB * L), TB * L)],
                codes_v.at[buf], semc[buf])

        def gather_copies(buf):
            return [
                pltpu.make_async_copy(
                    table_hbm.at[idx_v.at[buf]],
                    rows_v.at[buf],
                    semg[buf])
            ]

        def out_copies(b, buf):
            fb = blk0 + b
            n = fb // bt_n
            bt = fb % bt_n
            return [
                pltpu.make_async_copy(
                    stage_v.at[buf, pl.ds(dt * (8 * TB), 8 * TB)],
                    out_hbm.at[pl.ds(((n * L + dt) * bt_n + bt) * (8 * TB),
                                     8 * TB)],
                    semo[buf])
                for dt in range(D // 8)
            ]

        def compute_idx(buf):
            for i in range(TB * L // LANES):
                v = codes_v[buf, pl.ds(i * LANES, LANES)] + (i // 8) * VPAD
                idx_v[buf, pl.ds(i * LANES, LANES)] = v

        def accum_store(b, buf):
            for cp in out_copies(b, buf):
                cp.wait()

            @plsc.parallel_loop(0, TB, unroll=8)
            def tok_body(t):
                for h in range(D // 32):
                    r = [rows_v[buf, s * 128 + t, pl.ds(h * 32, 32)]
                         for s in range(L)]
                    while len(r) > 1:
                        r = [r[i] + r[i + 1] for i in range(0, len(r), 2)]
                    a = r[0]
                    even, odd = plsc.unpack(a, format=plsc.PackFormat.INTERLEAVED)
                    plsc.store_scatter(stage_v.at[buf], [bases[2 * h] + t], even)
                    plsc.store_scatter(stage_v.at[buf], [bases[2 * h + 1] + t], odd)
            for cp in out_copies(b, buf):
                cp.start()

        def phase(b, bufx, bufy):
            bp1 = jnp.minimum(b + 1, nb - 1)
            bp2 = jnp.minimum(b + 2, nb - 1)
            codes_copy(bp1, bufy).wait()
            compute_idx(bufy)
            for cp in gather_copies(bufy):
                cp.start()
            codes_copy(bp2, bufx).start()
            for cp in gather_copies(bufx):
                cp.wait()
            accum_store(b, bufx)

        codes_copy(0, 0).start()
        for buf in (0, 1):
            for cp in out_copies(buf, buf):
                cp.start()
        codes_copy(0, 0).wait()
        compute_idx(0)
        for cp in gather_copies(0):
            cp.start()
        codes_copy(1, 1).start()

        def pair_body(p, carry):
            phase(2 * p, 0, 1)
            phase(2 * p + 1, 1, 0)
            return carry

        lax.fori_loop(0, nb // 2, pair_body, 0)

        for cp in gather_copies(0):
            cp.wait()
        codes_copy(nb - 1, 1).wait()
        for buf in (0, 1):
            for cp in out_copies(nb - 2 + buf, buf):
                cp.wait()

    return sc_lookup


def kernel(codes, emb_table, W, b):
    bsz, nsz, lsz = codes.shape
    wr = W.reshape(D, L, RQ).transpose(1, 2, 0)
    emb_pad = jnp.pad(emb_table, ((0, VPAD - VOCAB), (0, 0)))
    table = _build_table(emb_pad, wr, b.reshape(1, D))
    codes_t = (codes.transpose(1, 2, 0)
               .reshape(nsz, lsz, bsz // TB, TB)
               .transpose(0, 2, 1, 3)
               .reshape(-1))
    out = _make_sc_lookup(bsz, nsz)(table, codes_t)
    out = (out.reshape(nsz, D // 8, bsz // TB, 8, TB)
           .transpose(2, 4, 0, 1, 3)
           .reshape(bsz, nsz, D))
    return out
